# baseline (speedup 1.0000x reference)
import functools

import jax
import jax.numpy as jnp
from jax import lax
from jax.experimental import pallas as pl
from jax.experimental.pallas import tpu as pltpu
from jax.experimental.pallas import tpu_sc as plsc

E = 160000
H = 256
NUM_CORES = 2
NUM_SUBCORES = 16
NW = NUM_CORES * NUM_SUBCORES
L = 16
CHUNK = 64
NCHUNKS = E // CHUNK
K = 80
KE = K * CHUNK
PAT = 5
NBLK = K // PAT

_mesh = plsc.VectorSubcoreMesh(core_axis_name="c", subcore_axis_name="s")


@functools.partial(
    pl.kernel,
    out_type=jax.ShapeDtypeStruct((E, H), jnp.float32),
    mesh=_mesh,
    compiler_params=pltpu.CompilerParams(needs_layout_passes=False),
    scratch_types=[
        pltpu.VMEM((KE,), jnp.int32),
        pltpu.VMEM((4, H), jnp.float32),
        pltpu.VMEM((CHUNK, H), jnp.float32),
        pltpu.VMEM((CHUNK, H), jnp.float32),
        pltpu.VMEM((CHUNK, H), jnp.float32),
        pltpu.VMEM((CHUNK, H), jnp.float32),
        pltpu.SemaphoreType.DMA,
        pltpu.SemaphoreType.DMA,
        pltpu.SemaphoreType.DMA,
        pltpu.SemaphoreType.DMA,
        pltpu.SemaphoreType.DMA,
        pltpu.SemaphoreType.DMA,
    ],
)
def _embed(idx_hbm, w_hbm, out_hbm, idx_v, table_v, bb0, bb1, sb0, sb1,
           bw0, bw1, sg0, sg1, sw0, sw1):
    bb = (bb0, bb1)
    sb = (sb0, sb1)
    bwsem = (bw0, bw1)
    sgsem = (sg0, sg1)
    swsem = (sw0, sw1)

    wid = lax.axis_index("s") * NUM_CORES + lax.axis_index("c")
    base = wid * K
    nvalid = jnp.minimum(K, NCHUNKS - base)
    start_e = pl.multiple_of(jnp.minimum(base * CHUNK, E - KE), 8)
    loff_e = pl.multiple_of(base * CHUNK - start_e, 8)

    pltpu.sync_copy(idx_hbm.at[pl.ds(start_e, KE)], idx_v)
    pltpu.sync_copy(w_hbm, table_v)

    lanes = lax.iota(jnp.int32, L)
    lane_of = [jnp.full((L,), j, jnp.int32) for j in range(L)]

    def idx_slice(t):
        return idx_v.at[pl.ds(pl.multiple_of(loff_e + t * CHUNK, 8), CHUNK)]

    def out_slice(t):
        return out_hbm.at[pl.ds(pl.multiple_of((base + t) * CHUNK, 8), CHUNK)]

    def bw_start(t, b):
        pltpu.make_async_copy(bb[b], out_slice(t), bwsem[b]).start()

    def bw_wait(t, b):
        pltpu.make_async_copy(bb[b], out_slice(t), bwsem[b]).wait()

    def sg_start(t, p):
        pltpu.make_async_copy(w_hbm.at[idx_slice(t)], sb[p], sgsem[p]).start()

    def sg_wait(t, p):
        pltpu.make_async_copy(w_hbm.at[idx_slice(t)], sb[p], sgsem[p]).wait()

    def sw_start(t, p):
        pltpu.make_async_copy(sb[p], out_slice(t), swsem[p]).start()

    def sw_wait(t, p):
        pltpu.make_async_copy(sb[p], out_slice(t), swsem[p]).wait()

    def build_chunk(t, buf):
        def rg_body(rg, carry):
            pos16 = pl.multiple_of(loff_e + t * CHUNK + rg * L, 8)
            iv16 = idx_v[pl.ds(pos16, L)]
            for j in range(L):
                ivj = iv16.at[lane_of[j]].get(
                    mode="promise_in_bounds")
                rowv = jnp.full((L,), rg * L + j, jnp.int32)
                for cg in range(H // L):
                    colv = lanes + cg * L
                    v = plsc.load_gather(table_v, [ivj, colv])
                    plsc.store_scatter(buf, [rowv, colv], v)
            return carry

        lax.fori_loop(0, CHUNK // L, rg_body, 0)

    def body(z2, carry):
        for zz in range(2):
            z = z2 * 2 + zz
            t0 = z * PAT

            for q in range(PAT - 1):
                t = t0 + q
                prev = t - 3 if q < 2 else t - 2
                b = q % 2

                @pl.when(t < nvalid)
                def _():
                    @pl.when(prev >= 0)
                    def _():
                        bw_wait(prev, b)
                    build_chunk(t, bb[b])
                    bw_start(t, b)

            ts = t0 + PAT - 1
            pz = zz
            ts_prev = ts - PAT

            @pl.when((ts_prev >= 0) & (ts_prev < nvalid))
            def _():
                sg_wait(ts_prev, 1 - pz)
                sw_start(ts_prev, 1 - pz)

            @pl.when(ts < nvalid)
            def _():
                @pl.when(ts - 2 * PAT >= 0)
                def _():
                    sw_wait(ts - 2 * PAT, pz)
                sg_start(ts, pz)

        return carry

    lax.fori_loop(0, NBLK // 2, body, 0)

    ts_last = NBLK * PAT - 1

    @pl.when(ts_last < nvalid)
    def _():
        sg_wait(ts_last, (NBLK - 1) % 2)
        sw_start(ts_last, (NBLK - 1) % 2)

    for p in range(2):
        bw_wait(0, p)
        sw_wait(0, p)


def kernel(edge_attr, weight):
    return _embed(edge_attr.astype(jnp.int32), weight.astype(jnp.float32))


# speedup vs baseline: 1.0163x; 1.0163x over previous
import functools

import jax
import jax.numpy as jnp
from jax import lax
from jax.experimental import pallas as pl
from jax.experimental.pallas import tpu as pltpu
from jax.experimental.pallas import tpu_sc as plsc

E = 160000
H = 256
NUM_CORES = 2
NUM_SUBCORES = 16
NW = NUM_CORES * NUM_SUBCORES
L = 16
CHUNK = 64
NCHUNKS = E // CHUNK
K = 80
KE = K * CHUNK
PAT = 5
NBLK = K // PAT

_mesh = plsc.VectorSubcoreMesh(core_axis_name="c", subcore_axis_name="s")


@functools.partial(
    pl.kernel,
    out_type=jax.ShapeDtypeStruct((E, H), jnp.float32),
    mesh=_mesh,
    compiler_params=pltpu.CompilerParams(needs_layout_passes=False),
    scratch_types=[
        pltpu.VMEM((KE,), jnp.int32),
        pltpu.VMEM((4, H), jnp.float32),
        pltpu.VMEM((CHUNK, H), jnp.float32),
        pltpu.VMEM((CHUNK, H), jnp.float32),
        pltpu.VMEM((CHUNK, H), jnp.float32),
        pltpu.VMEM((CHUNK, H), jnp.float32),
        pltpu.SemaphoreType.DMA,
        pltpu.SemaphoreType.DMA,
        pltpu.SemaphoreType.DMA,
        pltpu.SemaphoreType.DMA,
        pltpu.SemaphoreType.DMA,
        pltpu.SemaphoreType.DMA,
    ],
)
def _embed(idx_hbm, w_hbm, out_hbm, idx_v, table_v, bb0, bb1, sb0, sb1,
           bw0, bw1, sg0, sg1, sw0, sw1):
    bb = (bb0, bb1)
    sb = (sb0, sb1)
    bwsem = (bw0, bw1)
    sgsem = (sg0, sg1)
    swsem = (sw0, sw1)

    wid = lax.axis_index("s") * NUM_CORES + lax.axis_index("c")
    base = wid * K
    nvalid = jnp.minimum(K, NCHUNKS - base)
    start_e = pl.multiple_of(jnp.minimum(base * CHUNK, E - KE), 8)
    loff_e = pl.multiple_of(base * CHUNK - start_e, 8)

    pltpu.sync_copy(idx_hbm.at[pl.ds(start_e, KE)], idx_v)
    pltpu.sync_copy(w_hbm, table_v)

    lanes = lax.iota(jnp.int32, L)
    lane_of = [jnp.full((L,), j, jnp.int32) for j in range(L)]

    def idx_slice(t):
        return idx_v.at[pl.ds(pl.multiple_of(loff_e + t * CHUNK, 8), CHUNK)]

    def out_slice(t):
        return out_hbm.at[pl.ds(pl.multiple_of((base + t) * CHUNK, 8), CHUNK)]

    def bw_start(t, b):
        pltpu.make_async_copy(bb[b], out_slice(t), bwsem[b]).start()

    def bw_wait(t, b):
        pltpu.make_async_copy(bb[b], out_slice(t), bwsem[b]).wait()

    def sg_start(t, p):
        pltpu.make_async_copy(w_hbm.at[idx_slice(t)], sb[p], sgsem[p]).start()

    def sg_wait(t, p):
        pltpu.make_async_copy(w_hbm.at[idx_slice(t)], sb[p], sgsem[p]).wait()

    def sw_start(t, p):
        pltpu.make_async_copy(sb[p], out_slice(t), swsem[p]).start()

    def sw_wait(t, p):
        pltpu.make_async_copy(sb[p], out_slice(t), swsem[p]).wait()

    def build_chunk(t, buf):
        def rg_body(rg, carry):
            pos16 = pl.multiple_of(loff_e + t * CHUNK + rg * L, 8)
            iv16 = idx_v[pl.ds(pos16, L)]
            for j in range(L):
                ivj = iv16.at[lane_of[j]].get(
                    mode="promise_in_bounds")
                rowv = jnp.full((L,), rg * L + j, jnp.int32)
                for cg in range(H // L):
                    colv = lanes + cg * L
                    v = plsc.load_gather(table_v, [ivj, colv])
                    plsc.store_scatter(buf, [rowv, colv], v)
            return carry

        lax.fori_loop(0, CHUNK // L, rg_body, 0)

    def body(z2, carry):
        for zz in range(2):
            z = z2 * 2 + zz
            t0 = z * PAT

            for q in range(PAT - 1):
                t = t0 + q
                prev = t - 3 if q < 2 else t - 2
                b = q % 2

                @pl.when(t < nvalid)
                def _():
                    @pl.when(prev >= 0)
                    def _():
                        bw_wait(prev, b)
                    build_chunk(t, bb[b])
                    bw_start(t, b)

            ts = t0 + PAT - 1
            pz = zz
            ts_prev = ts - PAT

            t_abs = ts - 2 * PAT

            @pl.when((t_abs >= 0) & (t_abs < nvalid))
            def _():
                sg_wait(t_abs, pz)
                sw_start(t_abs, pz)
                sw_wait(t_abs, pz)

            @pl.when(ts < nvalid)
            def _():
                sg_start(ts, pz)

        return carry

    lax.fori_loop(0, NBLK // 2, body, 0)

    for zz in range(2):
        t_abs = (NBLK + zz) * PAT + (PAT - 1) - 2 * PAT

        @pl.when((t_abs >= 0) & (t_abs < nvalid))
        def _():
            sg_wait(t_abs, zz)
            sw_start(t_abs, zz)
            sw_wait(t_abs, zz)

    for p in range(2):
        bw_wait(0, p)


def kernel(edge_attr, weight):
    return _embed(edge_attr.astype(jnp.int32), weight.astype(jnp.float32))


# speedup vs baseline: 1.3863x; 1.3640x over previous
import functools

import jax
import jax.numpy as jnp
from jax import lax
from jax.experimental import pallas as pl
from jax.experimental.pallas import tpu as pltpu
from jax.experimental.pallas import tpu_sc as plsc

E = 160000
H = 256
NUM_CORES = 2
NUM_SUBCORES = 16
NW = NUM_CORES * NUM_SUBCORES
L = 16
CHUNK = 64
NCHUNKS = E // CHUNK
PAT = 9
NBLK = 9
K = PAT * NBLK
KE = K * CHUNK

_mesh = plsc.VectorSubcoreMesh(core_axis_name="c", subcore_axis_name="s")


@functools.partial(
    pl.kernel,
    out_type=jax.ShapeDtypeStruct((E, H), jnp.float32),
    mesh=_mesh,
    compiler_params=pltpu.CompilerParams(needs_layout_passes=False),
    scratch_types=[
        pltpu.VMEM((KE,), jnp.int32),
        pltpu.VMEM((4, H), jnp.float32),
        pltpu.VMEM((CHUNK, H), jnp.float32),
        pltpu.VMEM((CHUNK, H), jnp.float32),
        pltpu.VMEM((CHUNK, H), jnp.float32),
        pltpu.SemaphoreType.DMA,
        pltpu.SemaphoreType.DMA,
        pltpu.SemaphoreType.DMA,
        pltpu.SemaphoreType.DMA,
    ],
)
def _embed(idx_hbm, w_hbm, out_hbm, idx_v, table_v, bb0, bb1, sb,
           bw0, bw1, sgs, sws):
    bb = (bb0, bb1)
    bwsem = (bw0, bw1)

    wid = lax.axis_index("s") * NUM_CORES + lax.axis_index("c")
    base = wid * K
    nvalid = jnp.minimum(K, NCHUNKS - base)
    start_e = pl.multiple_of(jnp.minimum(base * CHUNK, E - KE), 8)
    loff_e = pl.multiple_of(base * CHUNK - start_e, 8)

    pltpu.sync_copy(idx_hbm.at[pl.ds(start_e, KE)], idx_v)
    pltpu.sync_copy(w_hbm, table_v)

    lanes = lax.iota(jnp.int32, L)
    lane_of = [jnp.full((L,), j, jnp.int32) for j in range(L)]

    def idx_slice(t):
        return idx_v.at[pl.ds(pl.multiple_of(loff_e + t * CHUNK, 8), CHUNK)]

    def out_slice(t):
        return out_hbm.at[pl.ds(pl.multiple_of((base + t) * CHUNK, 8), CHUNK)]

    def bw_start(t, b):
        pltpu.make_async_copy(bb[b], out_slice(t), bwsem[b]).start()

    def bw_wait(t, b):
        pltpu.make_async_copy(bb[b], out_slice(t), bwsem[b]).wait()

    def sg_start(t):
        pltpu.make_async_copy(w_hbm.at[idx_slice(t)], sb, sgs).start()

    def sg_wait(t):
        pltpu.make_async_copy(w_hbm.at[idx_slice(t)], sb, sgs).wait()

    def sw_start(t):
        pltpu.make_async_copy(sb, out_slice(t), sws).start()

    def sw_wait(t):
        pltpu.make_async_copy(sb, out_slice(t), sws).wait()

    def build_chunk(t, buf):
        def rg_body(rg, carry):
            pos16 = pl.multiple_of(loff_e + t * CHUNK + rg * L, 8)
            iv16 = idx_v[pl.ds(pos16, L)]
            for j in range(L):
                ivj = iv16.at[lane_of[j]].get(
                    mode="promise_in_bounds")
                rowv = jnp.full((L,), rg * L + j, jnp.int32)
                for cg in range(H // L):
                    colv = lanes + cg * L
                    v = plsc.load_gather(table_v, [ivj, colv])
                    plsc.store_scatter(buf, [rowv, colv], v)
            return carry

        lax.fori_loop(0, CHUNK // L, rg_body, 0)

    def body(z, carry):
        t0 = z * PAT

        for q in range(PAT - 1):
            t = t0 + q
            prev = t - 3 if q < 2 else t - 2
            b = q % 2

            @pl.when(t < nvalid)
            def _():
                @pl.when(prev >= 0)
                def _():
                    bw_wait(prev, b)
                build_chunk(t, bb[b])
                bw_start(t, b)

        ts = t0 + PAT - 1
        ts_prev = ts - PAT

        @pl.when((ts_prev >= 0) & (ts_prev < nvalid))
        def _():
            sg_wait(ts_prev)
            sw_start(ts_prev)
            sw_wait(ts_prev)

        @pl.when(ts < nvalid)
        def _():
            sg_start(ts)

        return carry

    lax.fori_loop(0, NBLK, body, 0)

    ts_last = NBLK * PAT - 1

    @pl.when(ts_last < nvalid)
    def _():
        sg_wait(ts_last)
        sw_start(ts_last)
        sw_wait(ts_last)

    @pl.when(nvalid > 0)
    def _():
        for p in range(2):
            bw_wait(0, p)


def kernel(edge_attr, weight):
    return _embed(edge_attr.astype(jnp.int32), weight.astype(jnp.float32))


# speedup vs baseline: 1.5594x; 1.1249x over previous
import functools

import jax
import jax.numpy as jnp
from jax import lax
from jax.experimental import pallas as pl
from jax.experimental.pallas import tpu as pltpu
from jax.experimental.pallas import tpu_sc as plsc

E = 160000
H = 256
NUM_CORES = 2
NUM_SUBCORES = 16
NW = NUM_CORES * NUM_SUBCORES
L = 16
CHUNK = 64
NCHUNKS = E // CHUNK
PAT = 9
NBLK = 9
K = PAT * NBLK
KE = K * CHUNK

_mesh = plsc.VectorSubcoreMesh(core_axis_name="c", subcore_axis_name="s")


@functools.partial(
    pl.kernel,
    out_type=jax.ShapeDtypeStruct((E, H), jnp.float32),
    mesh=_mesh,
    compiler_params=pltpu.CompilerParams(needs_layout_passes=False),
    scratch_types=[
        pltpu.VMEM((KE,), jnp.int32),
        pltpu.VMEM((4, H), jnp.float32),
        pltpu.VMEM((CHUNK, H), jnp.float32),
        pltpu.VMEM((CHUNK, H), jnp.float32),
        pltpu.VMEM((CHUNK, H), jnp.float32),
        pltpu.SemaphoreType.DMA,
        pltpu.SemaphoreType.DMA,
        pltpu.SemaphoreType.DMA,
        pltpu.SemaphoreType.DMA,
    ],
)
def _embed(idx_hbm, w_hbm, out_hbm, idx_v, table_v, bb0, bb1, sb,
           bw0, bw1, sgs, sws):
    bb = (bb0, bb1)
    bwsem = (bw0, bw1)

    wid = lax.axis_index("s") * NUM_CORES + lax.axis_index("c")
    base = wid * K
    nvalid = jnp.minimum(K, NCHUNKS - base)
    start_e = pl.multiple_of(jnp.minimum(base * CHUNK, E - KE), 8)
    loff_e = pl.multiple_of(base * CHUNK - start_e, 8)

    pltpu.sync_copy(idx_hbm.at[pl.ds(start_e, KE)], idx_v)
    pltpu.sync_copy(w_hbm, table_v)

    lanes = lax.iota(jnp.int32, L)
    lane_of = [jnp.full((L,), j, jnp.int32) for j in range(L)]

    def idx_slice(t):
        return idx_v.at[pl.ds(pl.multiple_of(loff_e + t * CHUNK, 8), CHUNK)]

    def out_slice(t):
        return out_hbm.at[pl.ds(pl.multiple_of((base + t) * CHUNK, 8), CHUNK)]

    def bw_start(t, b):
        pltpu.make_async_copy(bb[b], out_slice(t), bwsem[b]).start()

    def bw_wait(t, b):
        pltpu.make_async_copy(bb[b], out_slice(t), bwsem[b]).wait()

    def sg_start(t):
        pltpu.make_async_copy(w_hbm.at[idx_slice(t)], sb, sgs).start()

    def sg_wait(t):
        pltpu.make_async_copy(w_hbm.at[idx_slice(t)], sb, sgs).wait()

    def sw_start(t):
        pltpu.make_async_copy(sb, out_slice(t), sws).start()

    def sw_wait(t):
        pltpu.make_async_copy(sb, out_slice(t), sws).wait()

    def build_chunk(t, buf):
        def rg_body(rg, carry):
            pos16 = pl.multiple_of(loff_e + t * CHUNK + rg * L, 8)
            iv16 = idx_v[pl.ds(pos16, L)]
            for j in range(L):
                ivj = iv16.at[lane_of[j]].get(
                    mode="promise_in_bounds")
                rowv = jnp.full((L,), rg * L + j, jnp.int32)
                for cg in range(H // L):
                    colv = lanes + cg * L
                    v = plsc.load_gather(table_v, [ivj, colv])
                    plsc.store_scatter(buf, [rowv, colv], v)
            return carry

        lax.fori_loop(0, CHUNK // L, rg_body, 0)

    def body(i, carry):
        for b in range(2):
            t = i * 2 + b
            prev = t - 2

            @pl.when(t < nvalid)
            def _():
                @pl.when(prev >= 0)
                def _():
                    bw_wait(prev, b)
                build_chunk(t, bb[b])
                bw_start(t, b)

        return carry

    lax.fori_loop(0, (K + 1) // 2, body, 0)

    @pl.when(nvalid > 0)
    def _():
        for p in range(2):
            bw_wait(0, p)


def kernel(edge_attr, weight):
    return _embed(edge_attr.astype(jnp.int32), weight.astype(jnp.float32))


# speedup vs baseline: 1.5969x; 1.0240x over previous
import functools

import jax
import jax.numpy as jnp
from jax import lax
from jax.experimental import pallas as pl
from jax.experimental.pallas import tpu as pltpu
from jax.experimental.pallas import tpu_sc as plsc

E = 160000
H = 256
NUM_CORES = 2
NUM_SUBCORES = 16
NW = NUM_CORES * NUM_SUBCORES
L = 16
CHUNK = 64
NCHUNKS = E // CHUNK
K = -(-NCHUNKS // NW)
KE = K * CHUNK

_mesh = plsc.VectorSubcoreMesh(core_axis_name="c", subcore_axis_name="s")


@functools.partial(
    pl.kernel,
    out_type=jax.ShapeDtypeStruct((E, H), jnp.float32),
    mesh=_mesh,
    compiler_params=pltpu.CompilerParams(needs_layout_passes=False),
    scratch_types=[
        pltpu.VMEM((KE,), jnp.int32),
        pltpu.VMEM((4, H), jnp.float32),
        pltpu.VMEM((CHUNK, H), jnp.float32),
        pltpu.VMEM((CHUNK, H), jnp.float32),
        pltpu.VMEM((CHUNK, H), jnp.float32),
        pltpu.SemaphoreType.DMA,
        pltpu.SemaphoreType.DMA,
        pltpu.SemaphoreType.DMA,
        pltpu.SemaphoreType.DMA,
    ],
)
def _embed(idx_hbm, w_hbm, out_hbm, idx_v, table_v, bb0, bb1, sb,
           bw0, bw1, sgs, sws):
    bb = (bb0, bb1)
    bwsem = (bw0, bw1)

    wid = lax.axis_index("s") * NUM_CORES + lax.axis_index("c")
    base = wid * K
    nvalid = jnp.minimum(K, NCHUNKS - base)
    start_e = pl.multiple_of(jnp.minimum(base * CHUNK, E - KE), 8)
    loff_e = pl.multiple_of(base * CHUNK - start_e, 8)

    pltpu.sync_copy(idx_hbm.at[pl.ds(start_e, KE)], idx_v)
    pltpu.sync_copy(w_hbm, table_v)

    lanes = lax.iota(jnp.int32, L)
    lane_of = [jnp.full((L,), j, jnp.int32) for j in range(L)]

    def idx_slice(t):
        return idx_v.at[pl.ds(pl.multiple_of(loff_e + t * CHUNK, 8), CHUNK)]

    def out_slice(t):
        return out_hbm.at[pl.ds(pl.multiple_of((base + t) * CHUNK, 8), CHUNK)]

    def bw_start(t, b):
        pltpu.make_async_copy(bb[b], out_slice(t), bwsem[b]).start()

    def bw_wait(t, b):
        pltpu.make_async_copy(bb[b], out_slice(t), bwsem[b]).wait()

    def sg_start(t):
        pltpu.make_async_copy(w_hbm.at[idx_slice(t)], sb, sgs).start()

    def sg_wait(t):
        pltpu.make_async_copy(w_hbm.at[idx_slice(t)], sb, sgs).wait()

    def sw_start(t):
        pltpu.make_async_copy(sb, out_slice(t), sws).start()

    def sw_wait(t):
        pltpu.make_async_copy(sb, out_slice(t), sws).wait()

    def build_chunk(t, buf):
        def rg_body(rg, carry):
            pos16 = pl.multiple_of(loff_e + t * CHUNK + rg * L, 8)
            iv16 = idx_v[pl.ds(pos16, L)]
            for j in range(L):
                ivj = iv16.at[lane_of[j]].get(
                    mode="promise_in_bounds")
                rowv = jnp.full((L,), rg * L + j, jnp.int32)
                for cg in range(H // L):
                    colv = lanes + cg * L
                    v = plsc.load_gather(table_v, [ivj, colv])
                    plsc.store_scatter(buf, [rowv, colv], v)
            return carry

        lax.fori_loop(0, CHUNK // L, rg_body, 0)

    def body(i, carry):
        for b in range(2):
            t = i * 2 + b
            prev = t - 2

            @pl.when(t < nvalid)
            def _():
                @pl.when(prev >= 0)
                def _():
                    bw_wait(prev, b)
                build_chunk(t, bb[b])
                bw_start(t, b)

        return carry

    lax.fori_loop(0, (K + 1) // 2, body, 0)

    @pl.when(nvalid > 0)
    def _():
        for p in range(2):
            bw_wait(0, p)


def kernel(edge_attr, weight):
    return _embed(edge_attr.astype(jnp.int32), weight.astype(jnp.float32))
